# Initial kernel scaffold; baseline (speedup 1.0000x reference)
import jax
import jax.numpy as jnp
from jax.experimental import pallas as pl


def kernel(x, flow_patterns, W_sel, b_sel, W_int, b_int):
    raise NotImplementedError("write your pallas kernel here")


# speedup vs baseline: 29.8620x; 29.8620x over previous
import functools

import jax
import jax.numpy as jnp
from jax.experimental import pallas as pl

_SPARSITY = 0.1


def _fused_body(x_ref, wselT_ref, bsel_ref, wintT_ref, bint_ref, pat_ref,
                out_ref, ent_ref, inten_ref, pwsum_ref, *, k):
    i = pl.program_id(0)
    x = x_ref[...]
    xb = x.astype(jnp.bfloat16)

    logits = jnp.dot(xb, wselT_ref[...].astype(jnp.bfloat16),
                     preferred_element_type=jnp.float32) + bsel_ref[...]
    m = jnp.max(logits, axis=-1, keepdims=True)
    e = jnp.exp(logits - m)
    pw = e / jnp.sum(e, axis=-1, keepdims=True)

    inten = jax.nn.sigmoid(
        jnp.dot(xb, wintT_ref[...].astype(jnp.bfloat16),
                preferred_element_type=jnp.float32)
        + bint_ref[...])

    flow = jnp.dot(pw.astype(jnp.bfloat16),
                   pat_ref[...].astype(jnp.bfloat16),
                   preferred_element_type=jnp.float32) * inten

    keys = jax.lax.bitcast_convert_type(flow, jnp.int32) & jnp.int32(0x7FFFFFFF)

    t = keys.shape[0]
    lo = jnp.zeros((t, 1), jnp.int32)
    hi = jnp.full((t, 1), jnp.int32(0x7F800001))

    def search_step(_, carry):
        lo, hi = carry
        mid = lo + ((hi - lo) >> 1)
        cnt = jnp.sum((keys >= mid).astype(jnp.int32), axis=1, keepdims=True)
        ge = cnt >= k
        return jnp.where(ge, mid, lo), jnp.where(ge, hi, mid)

    lo, hi = jax.lax.fori_loop(0, 31, search_step, (lo, hi))

    out_ref[...] = jnp.where(keys >= lo, flow, 0.0)

    ent_blk = -jnp.sum(pw * jnp.log(pw + 1e-8), axis=(0, 1), keepdims=True)
    int_blk = jnp.sum(inten, axis=(0, 1), keepdims=True)
    pw_blk = jnp.sum(pw, axis=0, keepdims=True)

    @pl.when(i == 0)
    def _init():
        ent_ref[...] = jnp.zeros_like(ent_ref)
        inten_ref[...] = jnp.zeros_like(inten_ref)
        pwsum_ref[...] = jnp.zeros_like(pwsum_ref)

    ent_ref[...] += ent_blk
    inten_ref[...] += int_blk
    pwsum_ref[...] += pw_blk


def kernel(x, flow_patterns, W_sel, b_sel, W_int, b_int):
    B, S, IN = x.shape
    P, OUT, _ = flow_patterns.shape
    BS = B * S
    k = max(1, int(OUT * IN * _SPARSITY))
    T = 128
    grid = BS // T

    x2 = x.reshape(BS, IN)
    pat = flow_patterns.reshape(P, OUT * IN)
    wselT = W_sel.T
    bsel = b_sel.reshape(1, P)
    wintT = W_int.T
    bint = b_int.reshape(1, 1)

    out_flat, ent, inten, pwsum = pl.pallas_call(
        functools.partial(_fused_body, k=k),
        grid=(grid,),
        in_specs=[
            pl.BlockSpec((T, IN), lambda i: (i, 0)),
            pl.BlockSpec((IN, P), lambda i: (0, 0)),
            pl.BlockSpec((1, P), lambda i: (0, 0)),
            pl.BlockSpec((IN, 1), lambda i: (0, 0)),
            pl.BlockSpec((1, 1), lambda i: (0, 0)),
            pl.BlockSpec((P, OUT * IN), lambda i: (0, 0)),
        ],
        out_specs=[
            pl.BlockSpec((T, OUT * IN), lambda i: (i, 0)),
            pl.BlockSpec((1, 1), lambda i: (0, 0)),
            pl.BlockSpec((1, 1), lambda i: (0, 0)),
            pl.BlockSpec((1, P), lambda i: (0, 0)),
        ],
        out_shape=[
            jax.ShapeDtypeStruct((BS, OUT * IN), jnp.float32),
            jax.ShapeDtypeStruct((1, 1), jnp.float32),
            jax.ShapeDtypeStruct((1, 1), jnp.float32),
            jax.ShapeDtypeStruct((1, P), jnp.float32),
        ],
    )(x2, wselT, bsel, wintT, bint, pat)

    out = out_flat.reshape(B, S, OUT, IN)
    pattern_entropy = (ent[0, 0] / BS).astype(jnp.float32)
    flow_intensity_mean = (inten[0, 0] / BS).astype(jnp.float32)
    mvec = pwsum[0] / BS
    mu = jnp.mean(mvec)
    pattern_diversity = jnp.sqrt(jnp.sum((mvec - mu) ** 2) / (P - 1))
    return (out, pattern_entropy, flow_intensity_mean, pattern_diversity)
